# Initial kernel scaffold; baseline (speedup 1.0000x reference)
import jax
import jax.numpy as jnp
from jax.experimental import pallas as pl


def kernel(state_idx, character, monsters, card_idx, card_scalars, energy, card_table, state_table, W1, b1, W2, b2, W3, b3):
    raise NotImplementedError("write your pallas kernel here")


# speedup vs baseline: 3.2461x; 3.2461x over previous
import functools

import jax
import jax.numpy as jnp
from jax import lax
from jax.experimental import pallas as pl
from jax.experimental.pallas import tpu as pltpu
from jax.experimental.pallas import tpu_sc as plsc

B = 16384
H = 10
NCARD = 5
NSTATE = 3
CARD_EMB = 64
STATE_EMB = 32
HID = 64
OUT_DIM = 12
DENSE_IN = 3 + 6 + 2 * H + 2

NC, NS = 2, 16
NW = NC * NS
BPW = B // NW
GRP = BPW // 16


def _precompute_body(ct_ref, st_ref, w1e_ref, w1s_ref, p_ref):
    ct = ct_ref[...]
    rid = lax.broadcasted_iota(jnp.int32, ct.shape, 0)
    ct0 = jnp.where(rid == 0, 0.0, ct)
    parts = [jnp.dot(st_ref[...], w1s_ref[...], preferred_element_type=jnp.float32)]
    for h in range(H):
        parts.append(jnp.dot(ct0, w1e_ref[h], preferred_element_type=jnp.float32))
    parts.append(jnp.zeros((64 - NSTATE - H * NCARD, 64), jnp.float32))
    p_ref[...] = jnp.concatenate(parts, axis=0)


def _precompute_P(card_table, state_table, w1_emb, w1_state):
    return pl.pallas_call(
        _precompute_body,
        out_shape=jax.ShapeDtypeStruct((64, 64), jnp.float32),
    )(card_table, state_table, w1_emb, w1_state)


def _bag_body(p_hbm, sidx_hbm, cidx_hbm, out_hbm, p_v, sidx_v, cidx_v, out_v):
    wid = lax.axis_index("s") * NC + lax.axis_index("c")
    base = wid * BPW
    pltpu.sync_copy(p_hbm, p_v)
    pltpu.sync_copy(sidx_hbm.at[pl.ds(base, BPW)], sidx_v)
    for h in range(H):
        pltpu.sync_copy(cidx_hbm.at[pl.ds(h * B + base, BPW)], cidx_v.at[h])

    lane = lax.iota(jnp.int32, 16)

    def group(g, _):
        s = g * 16
        rows = [sidx_v[pl.ds(s, 16)] * 64]
        for h in range(H):
            rows.append((cidx_v[h, pl.ds(s, 16)] + (NSTATE + NCARD * h)) * 64)
        obase = (lane + s) * 64
        for d in range(64):
            acc = plsc.load_gather(p_v, [rows[0] + d])
            for r in rows[1:]:
                acc = acc + plsc.load_gather(p_v, [r + d])
            plsc.store_scatter(out_v, [obase + d], acc)
        return 0

    lax.fori_loop(0, GRP, group, 0)
    pltpu.sync_copy(out_v, out_hbm.at[pl.ds(base * 64, BPW * 64)])


@functools.lru_cache(maxsize=1)
def _make_bag_kernel():
    return functools.partial(
        pl.kernel,
        out_type=jax.ShapeDtypeStruct((B * 64,), jnp.float32),
        mesh=plsc.VectorSubcoreMesh(core_axis_name="c", subcore_axis_name="s"),
        compiler_params=pltpu.CompilerParams(needs_layout_passes=False),
        scratch_types=[
            pltpu.VMEM((64 * 64,), jnp.float32),
            pltpu.VMEM((BPW,), jnp.int32),
            pltpu.VMEM((H, BPW), jnp.int32),
            pltpu.VMEM((BPW * 64,), jnp.float32),
        ],
    )(_bag_body)


def _mlp_body(ch_ref, mo_ref, cs_ref, en_ref, bag_ref,
              w1_ref, b1_ref, w2_ref, b2_ref, w3_ref, b3_ref, out_ref):
    x = jnp.concatenate(
        [ch_ref[...], mo_ref[...], cs_ref[...], en_ref[...]], axis=1)
    h1 = jnp.dot(x, w1_ref[...], preferred_element_type=jnp.float32)
    h1 = jnp.maximum(h1 + bag_ref[...] + b1_ref[...], 0.0)
    h2 = jnp.dot(h1, w2_ref[...], preferred_element_type=jnp.float32)
    h2 = jnp.maximum(h2 + b2_ref[...], 0.0)
    out_ref[...] = jnp.dot(h2, w3_ref[...],
                           preferred_element_type=jnp.float32) + b3_ref[...]


def _mlp(character, monsters, cs2, energy, bag, w1d, b1, w2, b2, w3, b3,
         blk=2048):
    grid = (B // blk,)
    bspec = lambda w: pl.BlockSpec((blk, w), lambda i: (i, 0))
    wspec = lambda a, b: pl.BlockSpec((a, b), lambda i: (0, 0))
    return pl.pallas_call(
        _mlp_body,
        grid=grid,
        in_specs=[
            bspec(3), bspec(6), bspec(2 * H), bspec(2), bspec(64),
            wspec(DENSE_IN, HID), wspec(1, HID),
            wspec(HID, HID), wspec(1, HID),
            wspec(HID, OUT_DIM), wspec(1, OUT_DIM),
        ],
        out_specs=bspec(OUT_DIM),
        out_shape=jax.ShapeDtypeStruct((B, OUT_DIM), jnp.float32),
    )(character, monsters, cs2, energy, bag, w1d, b1, w2, b2, w3, b3)


@jax.jit
def kernel(state_idx, character, monsters, card_idx, card_scalars, energy,
           card_table, state_table, W1, b1, W2, b2, W3, b3):
    w1_state = W1[0:STATE_EMB]
    w1_emb = jnp.stack(
        [W1[43 + 66 * h: 43 + 66 * h + CARD_EMB] for h in range(H)])
    dense_rows = ([32, 33, 34, 35, 36, 37, 38, 39, 40]
                  + [41 + 66 * h + s for h in range(H) for s in range(2)]
                  + [701, 702])
    w1d = W1[jnp.array(dense_rows)]

    P = _precompute_P(card_table, state_table, w1_emb, w1_state)

    bag = _make_bag_kernel()(
        P.reshape(-1),
        state_idx.astype(jnp.int32),
        jnp.transpose(card_idx.astype(jnp.int32)).reshape(-1)).reshape(B, 64)

    cs2 = card_scalars.reshape(B, 2 * H)
    out = _mlp(character, monsters, cs2, energy, bag,
               w1d, b1.reshape(1, HID), W2, b2.reshape(1, HID),
               W3, b3.reshape(1, OUT_DIM))
    return out


# speedup vs baseline: 6.2421x; 1.9230x over previous
import functools

import jax
import jax.numpy as jnp
from jax import lax
from jax.experimental import pallas as pl
from jax.experimental.pallas import tpu as pltpu
from jax.experimental.pallas import tpu_sc as plsc

B = 16384
H = 10
NCARD = 5
NSTATE = 3
CARD_EMB = 64
STATE_EMB = 32
HID = 64
OUT_DIM = 12
DENSE_IN = 3 + 6 + 2 * H + 2

NC, NS = 2, 16
NW = NC * NS
BPW = B // NW
GRP = BPW // 16
STR = 65


def _precompute_body(ct_ref, st_ref, w1e_ref, w1s_ref, p_ref):
    ct = ct_ref[...]
    rid = lax.broadcasted_iota(jnp.int32, ct.shape, 0)
    ct0 = jnp.where(rid == 0, 0.0, ct)
    parts = [jnp.dot(st_ref[...], w1s_ref[...], preferred_element_type=jnp.float32)]
    for h in range(H):
        parts.append(jnp.dot(ct0, w1e_ref[h], preferred_element_type=jnp.float32))
    parts.append(jnp.zeros((64 - NSTATE - H * NCARD, 64), jnp.float32))
    p_ref[...] = jnp.concatenate(parts, axis=0)


def _precompute_P(card_table, state_table, w1_emb, w1_state):
    return pl.pallas_call(
        _precompute_body,
        out_shape=jax.ShapeDtypeStruct((64, 64), jnp.float32),
    )(card_table, state_table, w1_emb, w1_state)


def _bag_body(p_hbm, sidx_hbm, cidx_hbm, out_hbm, p_v, sidx_v, cidx_v, out_v):
    wid = lax.axis_index("s") * NC + lax.axis_index("c")
    base = wid * BPW
    pltpu.sync_copy(p_hbm, p_v)
    pltpu.sync_copy(sidx_hbm.at[pl.ds(base, BPW)], sidx_v)
    for h in range(H):
        pltpu.sync_copy(cidx_hbm.at[pl.ds(h * B + base, BPW)], cidx_v.at[h])

    lane = lax.iota(jnp.int32, 16)

    def group(g, _):
        s = g * 16
        rows = [sidx_v[pl.ds(s, 16)] * STR]
        for h in range(H):
            rows.append((cidx_v[h, pl.ds(s, 16)] + (NSTATE + NCARD * h)) * STR)
        obase = (lane + s) * STR
        for d in range(64):
            acc = plsc.load_gather(p_v, [rows[0] + d])
            for r in rows[1:]:
                acc = acc + plsc.load_gather(p_v, [r + d])
            plsc.store_scatter(out_v, [obase + d], acc)
        return 0

    lax.fori_loop(0, GRP, group, 0)
    pltpu.sync_copy(out_v, out_hbm.at[pl.ds(base * STR, BPW * STR)])


@functools.lru_cache(maxsize=1)
def _make_bag_kernel():
    return functools.partial(
        pl.kernel,
        out_type=jax.ShapeDtypeStruct((B * STR,), jnp.float32),
        mesh=plsc.VectorSubcoreMesh(core_axis_name="c", subcore_axis_name="s"),
        compiler_params=pltpu.CompilerParams(needs_layout_passes=False),
        scratch_types=[
            pltpu.VMEM((64 * STR,), jnp.float32),
            pltpu.VMEM((BPW,), jnp.int32),
            pltpu.VMEM((H, BPW), jnp.int32),
            pltpu.VMEM((BPW * STR,), jnp.float32),
        ],
    )(_bag_body)


def _mlp_body(ch_ref, mo_ref, cs_ref, en_ref, bag_ref,
              w1_ref, b1_ref, w2_ref, b2_ref, w3_ref, b3_ref, out_ref):
    x = jnp.concatenate(
        [ch_ref[...], mo_ref[...], cs_ref[...], en_ref[...]], axis=1)
    h1 = jnp.dot(x, w1_ref[...], preferred_element_type=jnp.float32)
    h1 = jnp.maximum(h1 + bag_ref[:, :HID] + b1_ref[...], 0.0)
    h2 = jnp.dot(h1, w2_ref[...], preferred_element_type=jnp.float32)
    h2 = jnp.maximum(h2 + b2_ref[...], 0.0)
    out_ref[...] = jnp.dot(h2, w3_ref[...],
                           preferred_element_type=jnp.float32) + b3_ref[...]


def _mlp(character, monsters, cs2, energy, bag, w1d, b1, w2, b2, w3, b3,
         blk=2048):
    grid = (B // blk,)
    bspec = lambda w: pl.BlockSpec((blk, w), lambda i: (i, 0))
    wspec = lambda a, b: pl.BlockSpec((a, b), lambda i: (0, 0))
    return pl.pallas_call(
        _mlp_body,
        grid=grid,
        in_specs=[
            bspec(3), bspec(6), bspec(2 * H), bspec(2), bspec(STR),
            wspec(DENSE_IN, HID), wspec(1, HID),
            wspec(HID, HID), wspec(1, HID),
            wspec(HID, OUT_DIM), wspec(1, OUT_DIM),
        ],
        out_specs=bspec(OUT_DIM),
        out_shape=jax.ShapeDtypeStruct((B, OUT_DIM), jnp.float32),
    )(character, monsters, cs2, energy, bag, w1d, b1, w2, b2, w3, b3)


@jax.jit
def kernel(state_idx, character, monsters, card_idx, card_scalars, energy,
           card_table, state_table, W1, b1, W2, b2, W3, b3):
    w1_state = W1[0:STATE_EMB]
    w1_emb = jnp.stack(
        [W1[43 + 66 * h: 43 + 66 * h + CARD_EMB] for h in range(H)])
    dense_rows = ([32, 33, 34, 35, 36, 37, 38, 39, 40]
                  + [41 + 66 * h + s for h in range(H) for s in range(2)]
                  + [701, 702])
    w1d = W1[jnp.array(dense_rows)]

    P = _precompute_P(card_table, state_table, w1_emb, w1_state)

    bag = _make_bag_kernel()(
        jnp.pad(P, ((0, 0), (0, STR - 64))).reshape(-1),
        state_idx.astype(jnp.int32),
        jnp.transpose(card_idx.astype(jnp.int32)).reshape(-1)).reshape(B, STR)

    cs2 = card_scalars.reshape(B, 2 * H)
    out = _mlp(character, monsters, cs2, energy, bag,
               w1d, b1.reshape(1, HID), W2, b2.reshape(1, HID),
               W3, b3.reshape(1, OUT_DIM))
    return out


# speedup vs baseline: 7.1169x; 1.1401x over previous
import functools

import jax
import jax.numpy as jnp
from jax import lax
from jax.experimental import pallas as pl
from jax.experimental.pallas import tpu as pltpu
from jax.experimental.pallas import tpu_sc as plsc

B = 16384
H = 10
NCARD = 5
NSTATE = 3
CARD_EMB = 64
STATE_EMB = 32
HID = 64
OUT_DIM = 12
DENSE_IN = 3 + 6 + 2 * H + 2

NC, NS = 2, 16
NW = NC * NS
BPW = B // NW
GRP = BPW // 16


def _precompute_body(ct_ref, st_ref, w1e_ref, w1s_ref, p_ref):
    ct = ct_ref[...]
    rid = lax.broadcasted_iota(jnp.int32, ct.shape, 0)
    ct0 = jnp.where(rid == 0, 0.0, ct)
    parts = [jnp.dot(st_ref[...], w1s_ref[...], preferred_element_type=jnp.float32)]
    for h in range(H):
        parts.append(jnp.dot(ct0, w1e_ref[h], preferred_element_type=jnp.float32))
    parts.append(jnp.zeros((64 - NSTATE - H * NCARD, 64), jnp.float32))
    p_ref[...] = jnp.concatenate(parts, axis=0)


def _precompute_P(card_table, state_table, w1_emb, w1_state):
    return pl.pallas_call(
        _precompute_body,
        out_shape=jax.ShapeDtypeStruct((64, 64), jnp.float32),
    )(card_table, state_table, w1_emb, w1_state)


def _bag_body(p_hbm, sidx_hbm, cidx_hbm, out_hbm, p_v, sidx_v, cidx_v, out_v):
    wid = lax.axis_index("s") * NC + lax.axis_index("c")
    base = wid * BPW
    pltpu.sync_copy(p_hbm, p_v)
    pltpu.sync_copy(sidx_hbm.at[pl.ds(base, BPW)], sidx_v)
    for h in range(H):
        pltpu.sync_copy(cidx_hbm.at[pl.ds(h * B + base, BPW)], cidx_v.at[h])

    def group(g, _):
        s = g * 16
        rows = [sidx_v[pl.ds(s, 16)] * 64]
        for h in range(H):
            rows.append((cidx_v[h, pl.ds(s, 16)] + (NSTATE + NCARD * h)) * 64)
        for u in range(16):
            offs = [r[u] for r in rows]
            ob = (s + u) * 64
            for j in range(4):
                acc = p_v[pl.ds(offs[0] + j * 16, 16)]
                for o in offs[1:]:
                    acc = acc + p_v[pl.ds(o + j * 16, 16)]
                out_v[pl.ds(ob + j * 16, 16)] = acc
        return 0

    lax.fori_loop(0, GRP, group, 0)
    pltpu.sync_copy(out_v, out_hbm.at[pl.ds(base * 64, BPW * 64)])


@functools.lru_cache(maxsize=1)
def _make_bag_kernel():
    return functools.partial(
        pl.kernel,
        out_type=jax.ShapeDtypeStruct((B * 64,), jnp.float32),
        mesh=plsc.VectorSubcoreMesh(core_axis_name="c", subcore_axis_name="s"),
        compiler_params=pltpu.CompilerParams(needs_layout_passes=False),
        scratch_types=[
            pltpu.VMEM((64 * 64,), jnp.float32),
            pltpu.VMEM((BPW,), jnp.int32),
            pltpu.VMEM((H, BPW), jnp.int32),
            pltpu.VMEM((BPW * 64,), jnp.float32),
        ],
    )(_bag_body)


def _mlp_body(ch_ref, mo_ref, cs_ref, en_ref, bag_ref,
              w1_ref, b1_ref, w2_ref, b2_ref, w3_ref, b3_ref, out_ref):
    x = jnp.concatenate(
        [ch_ref[...], mo_ref[...], cs_ref[...], en_ref[...]], axis=1)
    h1 = jnp.dot(x, w1_ref[...], preferred_element_type=jnp.float32)
    h1 = jnp.maximum(h1 + bag_ref[...] + b1_ref[...], 0.0)
    h2 = jnp.dot(h1, w2_ref[...], preferred_element_type=jnp.float32)
    h2 = jnp.maximum(h2 + b2_ref[...], 0.0)
    out_ref[...] = jnp.dot(h2, w3_ref[...],
                           preferred_element_type=jnp.float32) + b3_ref[...]


def _mlp(character, monsters, cs2, energy, bag, w1d, b1, w2, b2, w3, b3,
         blk=2048):
    grid = (B // blk,)
    bspec = lambda w: pl.BlockSpec((blk, w), lambda i: (i, 0))
    wspec = lambda a, b: pl.BlockSpec((a, b), lambda i: (0, 0))
    return pl.pallas_call(
        _mlp_body,
        grid=grid,
        in_specs=[
            bspec(3), bspec(6), bspec(2 * H), bspec(2), bspec(64),
            wspec(DENSE_IN, HID), wspec(1, HID),
            wspec(HID, HID), wspec(1, HID),
            wspec(HID, OUT_DIM), wspec(1, OUT_DIM),
        ],
        out_specs=bspec(OUT_DIM),
        out_shape=jax.ShapeDtypeStruct((B, OUT_DIM), jnp.float32),
    )(character, monsters, cs2, energy, bag, w1d, b1, w2, b2, w3, b3)


@jax.jit
def kernel(state_idx, character, monsters, card_idx, card_scalars, energy,
           card_table, state_table, W1, b1, W2, b2, W3, b3):
    w1_state = W1[0:STATE_EMB]
    w1_emb = jnp.stack(
        [W1[43 + 66 * h: 43 + 66 * h + CARD_EMB] for h in range(H)])
    dense_rows = ([32, 33, 34, 35, 36, 37, 38, 39, 40]
                  + [41 + 66 * h + s for h in range(H) for s in range(2)]
                  + [701, 702])
    w1d = W1[jnp.array(dense_rows)]

    P = _precompute_P(card_table, state_table, w1_emb, w1_state)

    bag = _make_bag_kernel()(
        P.reshape(-1),
        state_idx.astype(jnp.int32),
        jnp.transpose(card_idx.astype(jnp.int32)).reshape(-1)).reshape(B, 64)

    cs2 = card_scalars.reshape(B, 2 * H)
    out = _mlp(character, monsters, cs2, energy, bag,
               w1d, b1.reshape(1, HID), W2, b2.reshape(1, HID),
               W3, b3.reshape(1, OUT_DIM))
    return out


# speedup vs baseline: 9.0254x; 1.2682x over previous
import functools

import jax
import jax.numpy as jnp
from jax import lax
from jax.experimental import pallas as pl
from jax.experimental.pallas import tpu as pltpu
from jax.experimental.pallas import tpu_sc as plsc

B = 16384
H = 10
NCARD = 5
NSTATE = 3
CARD_EMB = 64
STATE_EMB = 32
HID = 64
OUT_DIM = 12
DENSE_IN = 3 + 6 + 2 * H + 2

NC, NS = 2, 16
NW = NC * NS
BPW = B // NW
GRP = BPW // 16
STR = 65


P_ROWS = 560


def _precompute_body(ct_ref, st_ref, w1e_ref, w1s_ref, p_ref):
    ct = ct_ref[...]
    rid = lax.broadcasted_iota(jnp.int32, ct.shape, 0)
    ct0 = jnp.where(rid == 0, 0.0, ct)
    S = jnp.dot(st_ref[...], w1s_ref[...], preferred_element_type=jnp.float32)
    T = [jnp.dot(ct0, w1e_ref[h], preferred_element_type=jnp.float32)
         for h in range(H)]
    zpad = jnp.zeros((3, 64), jnp.float32)
    t1p, t4p, t7p = (jnp.concatenate([T[k], zpad], axis=0) for k in (1, 4, 7))
    t9p = jnp.concatenate([T[9], zpad], axis=0)
    for s in range(NSTATE):
        for c0 in range(NCARD):
            p_ref[pl.ds(s * 40 + c0 * 8, 8), :] = (
                t1p + S[s: s + 1] + T[0][c0: c0 + 1])
    for c2 in range(NCARD):
        for c3 in range(NCARD):
            p_ref[pl.ds(120 + c2 * 40 + c3 * 8, 8), :] = (
                t4p + T[2][c2: c2 + 1] + T[3][c3: c3 + 1])
    for c5 in range(NCARD):
        for c6 in range(NCARD):
            p_ref[pl.ds(320 + c5 * 40 + c6 * 8, 8), :] = (
                t7p + T[5][c5: c5 + 1] + T[6][c6: c6 + 1])
    for c8 in range(NCARD):
        p_ref[pl.ds(520 + c8 * 8, 8), :] = t9p + T[8][c8: c8 + 1]


def _precompute_P(card_table, state_table, w1_emb, w1_state):
    return pl.pallas_call(
        _precompute_body,
        out_shape=jax.ShapeDtypeStruct((P_ROWS, 64), jnp.float32),
    )(card_table, state_table, w1_emb, w1_state)


def _bag_body(p_hbm, sidx_hbm, cidx_hbm, out_hbm, p_v, sidx_v, cidx_v, out_v):
    wid = lax.axis_index("s") * NC + lax.axis_index("c")
    base = wid * BPW
    pltpu.sync_copy(p_hbm, p_v)
    pltpu.sync_copy(sidx_hbm.at[pl.ds(base, BPW)], sidx_v)
    for h in range(H):
        pltpu.sync_copy(cidx_hbm.at[pl.ds(h * B + base, BPW)], cidx_v.at[h])

    lane = lax.iota(jnp.int32, 16)

    def group(g, _):
        s = g * 16
        sv = sidx_v[pl.ds(s, 16)]
        cv = [cidx_v[h, pl.ds(s, 16)] for h in range(H)]
        rows = [
            (sv * 40 + cv[0] * 8 + cv[1]) * STR,
            (cv[2] * 40 + cv[3] * 8 + cv[4] + 120) * STR,
            (cv[5] * 40 + cv[6] * 8 + cv[7] + 320) * STR,
            (cv[8] * 8 + cv[9] + 520) * STR,
        ]
        obase = (lane + s) * STR
        for d in range(64):
            acc = plsc.load_gather(p_v, [rows[0] + d])
            for r in rows[1:]:
                acc = acc + plsc.load_gather(p_v, [r + d])
            plsc.store_scatter(out_v, [obase + d], acc)
        return 0

    lax.fori_loop(0, GRP, group, 0)
    pltpu.sync_copy(out_v, out_hbm.at[pl.ds(base * STR, BPW * STR)])


@functools.lru_cache(maxsize=1)
def _make_bag_kernel():
    return functools.partial(
        pl.kernel,
        out_type=jax.ShapeDtypeStruct((B * STR,), jnp.float32),
        mesh=plsc.VectorSubcoreMesh(core_axis_name="c", subcore_axis_name="s"),
        compiler_params=pltpu.CompilerParams(needs_layout_passes=False),
        scratch_types=[
            pltpu.VMEM((P_ROWS * STR,), jnp.float32),
            pltpu.VMEM((BPW,), jnp.int32),
            pltpu.VMEM((H, BPW), jnp.int32),
            pltpu.VMEM((BPW * STR,), jnp.float32),
        ],
    )(_bag_body)


def _mlp_body(ch_ref, mo_ref, cs_ref, en_ref, bag_ref,
              w1_ref, b1_ref, w2_ref, b2_ref, w3_ref, b3_ref, out_ref):
    x = jnp.concatenate(
        [ch_ref[...], mo_ref[...], cs_ref[...], en_ref[...]], axis=1)
    h1 = jnp.dot(x, w1_ref[...], preferred_element_type=jnp.float32)
    h1 = jnp.maximum(h1 + bag_ref[:, :HID] + b1_ref[...], 0.0)
    h2 = jnp.dot(h1, w2_ref[...], preferred_element_type=jnp.float32)
    h2 = jnp.maximum(h2 + b2_ref[...], 0.0)
    out_ref[...] = jnp.dot(h2, w3_ref[...],
                           preferred_element_type=jnp.float32) + b3_ref[...]


def _mlp(character, monsters, cs2, energy, bag, w1d, b1, w2, b2, w3, b3,
         blk=2048):
    grid = (B // blk,)
    bspec = lambda w: pl.BlockSpec((blk, w), lambda i: (i, 0))
    wspec = lambda a, b: pl.BlockSpec((a, b), lambda i: (0, 0))
    return pl.pallas_call(
        _mlp_body,
        grid=grid,
        in_specs=[
            bspec(3), bspec(6), bspec(2 * H), bspec(2), bspec(STR),
            wspec(DENSE_IN, HID), wspec(1, HID),
            wspec(HID, HID), wspec(1, HID),
            wspec(HID, OUT_DIM), wspec(1, OUT_DIM),
        ],
        out_specs=bspec(OUT_DIM),
        out_shape=jax.ShapeDtypeStruct((B, OUT_DIM), jnp.float32),
    )(character, monsters, cs2, energy, bag, w1d, b1, w2, b2, w3, b3)


@jax.jit
def kernel(state_idx, character, monsters, card_idx, card_scalars, energy,
           card_table, state_table, W1, b1, W2, b2, W3, b3):
    w1_state = W1[0:STATE_EMB]
    w1_emb = jnp.stack(
        [W1[43 + 66 * h: 43 + 66 * h + CARD_EMB] for h in range(H)])
    dense_rows = ([32, 33, 34, 35, 36, 37, 38, 39, 40]
                  + [41 + 66 * h + s for h in range(H) for s in range(2)]
                  + [701, 702])
    w1d = W1[jnp.array(dense_rows)]

    P = _precompute_P(card_table, state_table, w1_emb, w1_state)

    bag = _make_bag_kernel()(
        jnp.pad(P, ((0, 0), (0, STR - 64))).reshape(-1),
        state_idx.astype(jnp.int32),
        jnp.transpose(card_idx.astype(jnp.int32)).reshape(-1)).reshape(B, STR)

    cs2 = card_scalars.reshape(B, 2 * H)
    out = _mlp(character, monsters, cs2, energy, bag,
               w1d, b1.reshape(1, HID), W2, b2.reshape(1, HID),
               W3, b3.reshape(1, OUT_DIM))
    return out
